# baseline (speedup 1.0000x reference)
import functools

import jax
import jax.numpy as jnp
from jax.experimental import pallas as pl

_LS = 0.1
_PAD_IDX = 0
_TRG_FACTOR = 1.0 - _LS


def _biased_kl_block(pred_ref, trg_ref, btrg_ref, boff_ref, out_ref, *, w):
    j = pl.program_id(1)
    trg = trg_ref[...]
    btrg = btrg_ref[...]
    boff = boff_ref[...]

    v = w * pl.num_programs(1)
    base = jnp.float32(_LS / (v - 2))
    c1 = base * jnp.log(base)
    pad = trg == _PAD_IDX

    trg_ampl = jnp.float32(_TRG_FACTOR) * (1.0 - boff)
    off = jnp.float32(_TRG_FACTOR) * boff
    d_t = trg_ampl + jnp.where(btrg == trg, off, 0.0)
    d_b = base + off
    g_t = d_t * jnp.log(d_t)
    g_b = d_b * jnp.log(d_b)
    base_r = jnp.where(pad, 0.0, base)
    c1_r = jnp.where(pad, 0.0, c1)
    d_t = jnp.where(pad, 0.0, d_t)
    g_t = jnp.where(pad, 0.0, g_t)
    d_b = jnp.where(pad, 0.0, d_b)
    g_b = jnp.where(pad, 0.0, g_b)

    r = trg.shape[0]
    cw = 256 if w % 256 == 0 else w
    lane = jax.lax.broadcasted_iota(jnp.int32, (r, cw), 1)

    def _chunk(k, carry):
        c0 = pl.multiple_of(k * cw, 128)
        cols = pl.ds(c0, cw)
        p = pred_ref[:, cols]
        col = lane + (j * w + c0)
        o = c1_r - base_r * p
        o = jnp.where(col == btrg, g_b - d_b * p, o)
        o = jnp.where(col == trg, g_t - d_t * p, o)
        out_ref[:, cols] = o
        return carry

    jax.lax.fori_loop(0, w // cw, _chunk, 0)

    @pl.when(j == 0)
    def _fix_col0():
        d_0 = jnp.where(jnp.logical_or(btrg != _PAD_IDX, pad), 0.0, off)
        g_0 = jnp.where(d_0 > 0, d_0 * jnp.log(jnp.maximum(d_0, 1e-30)), 0.0)
        out_ref[:, 0:1] = g_0 - d_0 * pred_ref[:, 0:1]


def kernel(pred, trg, biased_trg, biased_offset):
    b, s, v = pred.shape
    n = b * s
    pred2 = pred.reshape(n, v)
    trg2 = trg.reshape(n, 1)
    btrg2 = biased_trg.reshape(n, 1)
    boff2 = biased_offset.reshape(n, 1)

    rblk = 64 if n % 64 == 0 else n
    wblk = 32000 if v % 32000 == 0 else v
    grid = (n // rblk, v // wblk)

    row_spec = pl.BlockSpec((rblk, 1), lambda i, j: (i, 0))
    return pl.pallas_call(
        functools.partial(_biased_kl_block, w=wblk),
        grid=grid,
        in_specs=[
            pl.BlockSpec((rblk, wblk), lambda i, j: (i, j)),
            row_spec,
            row_spec,
            row_spec,
        ],
        out_specs=pl.BlockSpec((rblk, wblk), lambda i, j: (i, j)),
        out_shape=jax.ShapeDtypeStruct((n, v), jnp.float32),
    )(pred2, trg2, btrg2, boff2)


# speedup vs baseline: 3.0010x; 3.0010x over previous
import functools

import jax
import jax.numpy as jnp
from jax.experimental import pallas as pl

_LS = 0.1
_PAD_IDX = 0
_TRG_FACTOR = 1.0 - _LS


def _biased_kl_block(pred_ref, trg_ref, btrg_ref, boff_ref, out_ref, *, w):
    j = pl.program_id(1)
    pred = pred_ref[...]
    trg = trg_ref[...]
    btrg = btrg_ref[...]
    boff = boff_ref[...]

    v = w * pl.num_programs(1)
    base = jnp.float32(_LS / (v - 2))
    c1 = base * jnp.log(base)
    pad = trg == _PAD_IDX

    trg_ampl = jnp.float32(_TRG_FACTOR) * (1.0 - boff)
    off = jnp.float32(_TRG_FACTOR) * boff
    d_t = trg_ampl + jnp.where(btrg == trg, off, 0.0)
    d_b = base + off
    g_t = d_t * jnp.log(d_t)
    g_b = d_b * jnp.log(d_b)
    base_r = jnp.where(pad, 0.0, base)
    c1_r = jnp.where(pad, 0.0, c1)
    d_t = jnp.where(pad, 0.0, d_t)
    g_t = jnp.where(pad, 0.0, g_t)
    d_b = jnp.where(pad, 0.0, d_b)
    g_b = jnp.where(pad, 0.0, g_b)

    r = pred.shape[0]
    col = jax.lax.broadcasted_iota(jnp.int32, (r, w), 1) + j * w
    m_b = col == btrg
    m_t = col == trg
    d = jnp.where(m_t, d_t, jnp.where(m_b, d_b, base_r))
    g = jnp.where(m_t, g_t, jnp.where(m_b, g_b, c1_r))
    out_ref[...] = g - d * pred

    @pl.when(j == 0)
    def _fix_col0():
        d_0 = jnp.where(jnp.logical_or(btrg != _PAD_IDX, pad), 0.0, off)
        g_0 = jnp.where(d_0 > 0, d_0 * jnp.log(jnp.maximum(d_0, 1e-30)), 0.0)
        out_ref[:, 0:1] = g_0 - d_0 * pred[:, 0:1]


def kernel(pred, trg, biased_trg, biased_offset):
    b, s, v = pred.shape
    n = b * s
    pred2 = pred.reshape(n, v)
    trg2 = trg.reshape(n, 1)
    btrg2 = biased_trg.reshape(n, 1)
    boff2 = biased_offset.reshape(n, 1)

    rblk = 64 if n % 64 == 0 else n
    wblk = 32000 if v % 32000 == 0 else v
    grid = (n // rblk, v // wblk)

    row_spec = pl.BlockSpec((rblk, 1), lambda i, j: (i, 0))
    return pl.pallas_call(
        functools.partial(_biased_kl_block, w=wblk),
        grid=grid,
        in_specs=[
            pl.BlockSpec((rblk, wblk), lambda i, j: (i, j)),
            row_spec,
            row_spec,
            row_spec,
        ],
        out_specs=pl.BlockSpec((rblk, wblk), lambda i, j: (i, j)),
        out_shape=jax.ShapeDtypeStruct((n, v), jnp.float32),
    )(pred2, trg2, btrg2, boff2)
